# baseline (speedup 1.0000x reference)
import functools

import jax
import jax.numpy as jnp
from jax import lax
from jax.experimental import pallas as pl
from jax.experimental.pallas import tpu as pltpu
from jax.experimental.pallas import tpu_sc as plsc

_K = 16


def _topk_body(n, ptsT_ref, coords_ref, idx_ref):
    b = pl.program_id(0)
    q = ptsT_ref[0]
    c = coords_ref[0]
    sq_c = jnp.sum(c * c, axis=0, keepdims=True)
    sq_q = jnp.sum(q * q, axis=1, keepdims=True)
    qb = q.astype(jnp.bfloat16).astype(jnp.float32)
    cb = c.astype(jnp.bfloat16).astype(jnp.float32)
    prod = (qb[:, 0:1] * cb[0:1, :] + qb[:, 1:2] * cb[1:2, :]
            + qb[:, 2:3] * cb[2:3, :])
    d = sq_q + sq_c - 2.0 * prod
    iota = lax.broadcasted_iota(jnp.int32, d.shape, 1)
    inf = jnp.float32(jnp.inf)
    cols = []
    for j in range(_K + 1):
        am = jnp.argmin(d, axis=1).astype(jnp.int32)[:, None]
        if j > 0:
            cols.append(am + b * n)
        if j < _K:
            d = jnp.where(iota == am, inf, d)
    idx_ref[0] = jnp.concatenate(cols, axis=1)


def _topk_call(ptsT, coords, bq=512):
    b, nq, _ = ptsT.shape
    n = coords.shape[2]
    return pl.pallas_call(
        functools.partial(_topk_body, n),
        grid=(b, nq // bq),
        in_specs=[
            pl.BlockSpec((1, bq, 3), lambda bi, i: (bi, i, 0)),
            pl.BlockSpec((1, 3, n), lambda bi, i: (bi, 0, 0)),
        ],
        out_specs=pl.BlockSpec((1, bq, _K), lambda bi, i: (bi, i, 0)),
        out_shape=jax.ShapeDtypeStruct((b, nq, _K), jnp.int32),
    )(ptsT, coords)


def _proj_body(x_ref, w1_ref, w2_ref, o1_ref, o2_ref):
    x = x_ref[0]
    o1_ref[0] = jnp.dot(x, w1_ref[...], preferred_element_type=jnp.float32)
    o2_ref[0] = jnp.dot(x, w2_ref[...], preferred_element_type=jnp.float32)


def _proj_call(ft, w1t, w2t):
    b, n, c = ft.shape
    co = w1t.shape[1]
    out = jax.ShapeDtypeStruct((b, n, co), jnp.float32)
    return pl.pallas_call(
        _proj_body,
        grid=(b,),
        in_specs=[
            pl.BlockSpec((1, n, c), lambda bi: (bi, 0, 0)),
            pl.BlockSpec((c, co), lambda bi: (0, 0)),
            pl.BlockSpec((c, co), lambda bi: (0, 0)),
        ],
        out_specs=[
            pl.BlockSpec((1, n, co), lambda bi: (bi, 0, 0)),
            pl.BlockSpec((1, n, co), lambda bi: (bi, 0, 0)),
        ],
        out_shape=[out, out],
    )(ft, w1t, w2t)


def _sc_gather_reduce(bn, c):
    nw = 32
    npw = bn // nw
    gp = 1024 // c
    ch = 64
    mesh = plsc.VectorSubcoreMesh(core_axis_name="c", subcore_axis_name="s")
    out_sds = jax.ShapeDtypeStruct((bn, c), jnp.float32)

    @functools.partial(
        pl.kernel,
        out_type=(out_sds, out_sds, out_sds),
        mesh=mesh,
        scratch_types=[
            pltpu.VMEM((npw * _K,), jnp.int32),
            pltpu.VMEM((gp * _K, c), jnp.float32),
            pltpu.VMEM((gp * _K, c), jnp.float32),
            pltpu.VMEM((ch, c), jnp.float32),
            pltpu.VMEM((ch, c), jnp.float32),
            pltpu.VMEM((ch, c), jnp.float32),
            pltpu.SemaphoreType.DMA,
            pltpu.SemaphoreType.DMA,
        ],
    )
    def kern(g_hbm, idx_hbm, m_hbm, s_hbm, q_hbm,
             idx_v, rows0_v, rows1_v, m_v, s_v, q_v, sem0, sem1):
        wid = lax.axis_index("c") * 16 + lax.axis_index("s")
        base_pt = wid * npw
        pltpu.sync_copy(idx_hbm.at[pl.ds(base_pt * _K, npw * _K)], idx_v)
        ngroups = ch // gp

        def src(ci, gi):
            off = (ci * ch + gi * gp) * _K
            return g_hbm.at[idx_v.at[pl.ds(off, gp * _K)]]

        def reduce_group(gi, rows_v):
            for p in range(gp):
                row = gi * gp + p
                for j in range(c // 16):
                    sl = pl.ds(j * 16, 16)
                    r = rows_v[p * _K, sl]
                    mx = r
                    sm = r
                    qq = r * r
                    for i in range(1, _K):
                        r = rows_v[p * _K + i, sl]
                        mx = jnp.maximum(mx, r)
                        sm = sm + r
                        qq = qq + r * r
                    m_v[row, sl] = mx
                    s_v[row, sl] = sm
                    q_v[row, sl] = qq

        def chunk_body(ci, carry):
            pltpu.async_copy(src(ci, 0), rows0_v, sem0)

            def pair_body(pi, carry2):
                g0 = 2 * pi
                pltpu.async_copy(src(ci, g0 + 1), rows1_v, sem1)
                pltpu.make_async_copy(src(ci, g0), rows0_v, sem0).wait()
                reduce_group(g0, rows0_v)

                @pl.when(g0 + 2 < ngroups)
                def _():
                    pltpu.async_copy(src(ci, g0 + 2), rows0_v, sem0)

                pltpu.make_async_copy(src(ci, g0 + 1), rows1_v, sem1).wait()
                reduce_group(g0 + 1, rows1_v)
                return carry2

            lax.fori_loop(0, ngroups // 2, pair_body, 0)
            out_off = base_pt + ci * ch
            pltpu.sync_copy(m_v, m_hbm.at[pl.ds(out_off, ch)])
            pltpu.sync_copy(s_v, s_hbm.at[pl.ds(out_off, ch)])
            pltpu.sync_copy(q_v, q_hbm.at[pl.ds(out_off, ch)])
            return carry

        lax.fori_loop(0, npw // ch, chunk_body, 0)

    return kern


def _stage_finish(h, mx, s, q, n):
    nk = float(n * _K)
    sum_h = jnp.sum(h, axis=0, keepdims=True)
    sum_h2 = jnp.sum(h * h, axis=0, keepdims=True)
    sum_s = jnp.sum(s, axis=0, keepdims=True)
    cross = jnp.sum(h * s, axis=0, keepdims=True)
    sum_q = jnp.sum(q, axis=0, keepdims=True)
    mean = (_K * sum_h + sum_s) / nk
    e2 = (_K * sum_h2 + 2.0 * cross + sum_q) / nk
    inv = lax.rsqrt(e2 - mean * mean + 1e-5)
    v = (h + mx - mean) * inv
    return jnp.where(v >= 0, v, 0.2 * v)


def _stage1_body(n, h_ref, m_ref, s_ref, q_ref, wb_ref, wd_ref,
                 x1_ref, g2_ref, h2_ref):
    x1 = _stage_finish(h_ref[0], m_ref[0], s_ref[0], q_ref[0], n)
    x1_ref[0] = x1
    g2_ref[0] = jnp.dot(x1, wb_ref[...], preferred_element_type=jnp.float32)
    h2_ref[0] = jnp.dot(x1, wd_ref[...], preferred_element_type=jnp.float32)


def _stage1_call(h1, m1, s1, q1, w2bt, w2dt):
    b, n, c = h1.shape
    c2 = w2bt.shape[1]
    arr = lambda cc: pl.BlockSpec((1, n, cc), lambda bi: (bi, 0, 0))
    wspec = pl.BlockSpec((c, c2), lambda bi: (0, 0))
    return pl.pallas_call(
        functools.partial(_stage1_body, n),
        grid=(b,),
        in_specs=[arr(c), arr(c), arr(c), arr(c), wspec, wspec],
        out_specs=[arr(c), arr(c2), arr(c2)],
        out_shape=[
            jax.ShapeDtypeStruct((b, n, c), jnp.float32),
            jax.ShapeDtypeStruct((b, n, c2), jnp.float32),
            jax.ShapeDtypeStruct((b, n, c2), jnp.float32),
        ],
    )(h1, m1, s1, q1, w2bt, w2dt)


def _final_body(n, h2_ref, m2_ref, s2_ref, q2_ref, ft_ref, x1_ref,
                wa_ref, wb_ref, wc_ref, out_ref):
    x2 = _stage_finish(h2_ref[0], m2_ref[0], s2_ref[0], q2_ref[0], n)
    y = (jnp.dot(ft_ref[0], wa_ref[...], preferred_element_type=jnp.float32)
         + jnp.dot(x1_ref[0], wb_ref[...], preferred_element_type=jnp.float32)
         + jnp.dot(x2, wc_ref[...], preferred_element_type=jnp.float32))
    m3 = jnp.mean(y, axis=0, keepdims=True)
    e2 = jnp.mean(y * y, axis=0, keepdims=True)
    inv = lax.rsqrt(e2 - m3 * m3 + 1e-5)
    z = (y - m3) * inv
    z = jnp.where(z >= 0, z, 0.2 * z)
    out_ref[0] = z.T


def _final_call(h2, m2, s2, q2, ft, x1, w3at, w3bt, w3ct):
    b, n, c2 = h2.shape
    c = ft.shape[2]
    arr = lambda cc: pl.BlockSpec((1, n, cc), lambda bi: (bi, 0, 0))
    wsp = lambda ci: pl.BlockSpec((ci, c), lambda bi: (0, 0))
    return pl.pallas_call(
        functools.partial(_final_body, n),
        grid=(b,),
        in_specs=[arr(c2), arr(c2), arr(c2), arr(c2), arr(c), arr(c),
                  wsp(c), wsp(c), wsp(c2)],
        out_specs=pl.BlockSpec((1, c, n), lambda bi: (bi, 0, 0)),
        out_shape=jax.ShapeDtypeStruct((b, c, n), jnp.float32),
    )(h2, m2, s2, q2, ft, x1, w3at, w3bt, w3ct)


def kernel(coords, features, W1, W2, W3):
    b, c, n = features.shape
    ptsT = jnp.swapaxes(coords, 1, 2)
    ft = jnp.swapaxes(features, 1, 2)
    w1bt = W1[:, c:].T
    w1dt = (W1[:, :c] - W1[:, c:]).T
    w2bt = W2[:, c:].T
    w2dt = (W2[:, :c] - W2[:, c:]).T
    w3at = W3[:, :c].T
    w3bt = W3[:, c:2 * c].T
    w3ct = W3[:, 2 * c:].T

    g1, h1 = _proj_call(ft, w1bt, w1dt)
    nh = n // 2
    sc1 = _sc_gather_reduce(nh, c)
    sc2 = _sc_gather_reduce(n, 2 * c)

    outs = []
    msq1 = {}
    idxs = {}
    gate = None
    for i in range(b):
        halves = []
        for h in range(2):
            pslice = ptsT[i:i + 1, h * nh:(h + 1) * nh]
            if h == 1 and gate is not None:
                pslice = pslice + gate * 0.0
            ih = _topk_call(pslice, coords[i:i + 1]).reshape(-1)
            halves.append((ih, sc1(g1[i], ih)))
        idxs[i] = jnp.concatenate([hv[0] for hv in halves])
        msq1[i] = [jnp.concatenate([hv[1][t] for hv in halves])
                   for t in range(3)]
        m1, s1, q1 = msq1[i]
        x1, g2, h2 = _stage1_call(h1[i:i + 1], m1[None], s1[None], q1[None],
                                  w2bt, w2dt)
        m2, s2, q2 = sc2(g2[0], idxs[i])
        gate = g2[0, 0, 0]
        outs.append(_final_call(h2, m2[None], s2[None], q2[None],
                                ft[i:i + 1], x1, w3at, w3bt, w3ct))
    return jnp.concatenate(outs, axis=0)


# speedup vs baseline: 1.1107x; 1.1107x over previous
import functools

import jax
import jax.numpy as jnp
from jax import lax
from jax.experimental import pallas as pl
from jax.experimental.pallas import tpu as pltpu
from jax.experimental.pallas import tpu_sc as plsc

_K = 16


def _topk_body(n, ptsT_ref, coords_ref, idx_ref):
    b = pl.program_id(0)
    q = ptsT_ref[0]
    c = coords_ref[0]
    sq_c = jnp.sum(c * c, axis=0, keepdims=True)
    sq_q = jnp.sum(q * q, axis=1, keepdims=True)
    qb = q.astype(jnp.bfloat16).astype(jnp.float32)
    cb = c.astype(jnp.bfloat16).astype(jnp.float32)
    prod = (qb[:, 0:1] * cb[0:1, :] + qb[:, 1:2] * cb[1:2, :]
            + qb[:, 2:3] * cb[2:3, :])
    d = sq_q + sq_c - 2.0 * prod
    iota = lax.broadcasted_iota(jnp.int32, d.shape, 1)
    inf = jnp.float32(jnp.inf)
    cols = []
    for j in range(_K + 1):
        am = jnp.argmin(d, axis=1).astype(jnp.int32)[:, None]
        if j > 0:
            cols.append(am + b * n)
        if j < _K:
            d = jnp.where(iota == am, inf, d)
    idx_ref[0] = jnp.concatenate(cols, axis=1)


def _topk_call(ptsT, coords, bq=512):
    b, nq, _ = ptsT.shape
    n = coords.shape[2]
    return pl.pallas_call(
        functools.partial(_topk_body, n),
        grid=(b, nq // bq),
        in_specs=[
            pl.BlockSpec((1, bq, 3), lambda bi, i: (bi, i, 0)),
            pl.BlockSpec((1, 3, n), lambda bi, i: (bi, 0, 0)),
        ],
        out_specs=pl.BlockSpec((1, bq, _K), lambda bi, i: (bi, i, 0)),
        out_shape=jax.ShapeDtypeStruct((b, nq, _K), jnp.int32),
    )(ptsT, coords)


def _proj_body(x_ref, w1_ref, w2_ref, o1_ref, o2_ref):
    x = x_ref[0]
    o1_ref[0] = jnp.dot(x, w1_ref[...], preferred_element_type=jnp.float32)
    o2_ref[0] = jnp.dot(x, w2_ref[...], preferred_element_type=jnp.float32)


def _proj_call(ft, w1t, w2t):
    b, n, c = ft.shape
    co = w1t.shape[1]
    return pl.pallas_call(
        _proj_body,
        grid=(b,),
        in_specs=[
            pl.BlockSpec((1, n, c), lambda bi: (bi, 0, 0)),
            pl.BlockSpec((c, co), lambda bi: (0, 0)),
            pl.BlockSpec((c, co), lambda bi: (0, 0)),
        ],
        out_specs=[
            pl.BlockSpec((1, n, co), lambda bi: (bi, 0, 0)),
            pl.BlockSpec((1, n, co), lambda bi: (bi, 0, 0)),
        ],
        out_shape=[jax.ShapeDtypeStruct((b, n, co), jnp.float32),
                   jax.ShapeDtypeStruct((b, n, co), jnp.float32)],
    )(ft, w1t, w2t)


def _sc_gather_reduce(bn, c):
    nw = 32
    npw = bn // nw
    gp = 8
    ch = 4096 // c
    mesh = plsc.VectorSubcoreMesh(core_axis_name="c", subcore_axis_name="s")
    out_sds = jax.ShapeDtypeStruct((bn, c), jnp.float32)

    @functools.partial(
        pl.kernel,
        out_type=(out_sds, out_sds, out_sds),
        mesh=mesh,
        scratch_types=[
            pltpu.VMEM((npw * _K,), jnp.int32),
            pltpu.VMEM((gp * _K, c), jnp.float32),
            pltpu.VMEM((gp * _K, c), jnp.float32),
            pltpu.VMEM((ch, c), jnp.float32),
            pltpu.VMEM((ch, c), jnp.float32),
            pltpu.VMEM((ch, c), jnp.float32),
            pltpu.SemaphoreType.DMA,
            pltpu.SemaphoreType.DMA,
        ],
    )
    def kern(g_hbm, idx_hbm, m_hbm, s_hbm, q_hbm,
             idx_v, rows0_v, rows1_v, m_v, s_v, q_v, sem0, sem1):
        wid = lax.axis_index("c") * 16 + lax.axis_index("s")
        base_pt = wid * npw
        pltpu.sync_copy(idx_hbm.at[pl.ds(base_pt * _K, npw * _K)], idx_v)
        ngroups = ch // gp

        def src(ci, gi):
            off = (ci * ch + gi * gp) * _K
            return g_hbm.at[idx_v.at[pl.ds(off, gp * _K)]]

        def reduce_group(gi, rows_v):
            for p in range(gp):
                row = gi * gp + p
                for j in range(c // 16):
                    sl = pl.ds(j * 16, 16)
                    r = rows_v[p * _K, sl]
                    mx = r
                    sm = r
                    qq = r * r
                    for i in range(1, _K):
                        r = rows_v[p * _K + i, sl]
                        mx = jnp.maximum(mx, r)
                        sm = sm + r
                        qq = qq + r * r
                    m_v[row, sl] = mx
                    s_v[row, sl] = sm
                    q_v[row, sl] = qq

        bufs = (rows0_v, rows1_v)
        sems = (sem0, sem1)

        def chunk_body(ci, carry):
            descs = [pltpu.async_copy(src(ci, 0), rows0_v, sem0)]
            for g in range(ngroups):
                if g + 1 < ngroups:
                    descs.append(pltpu.async_copy(
                        src(ci, g + 1), bufs[(g + 1) % 2], sems[(g + 1) % 2]))
                descs[g].wait()
                reduce_group(g, bufs[g % 2])
            out_off = base_pt + ci * ch
            pltpu.sync_copy(m_v, m_hbm.at[pl.ds(out_off, ch)])
            pltpu.sync_copy(s_v, s_hbm.at[pl.ds(out_off, ch)])
            pltpu.sync_copy(q_v, q_hbm.at[pl.ds(out_off, ch)])
            return carry

        lax.fori_loop(0, npw // ch, chunk_body, 0)

    return kern


def _stage_finish(h, mx, s, q, n):
    mx = mx.astype(jnp.float32)
    s = s.astype(jnp.float32)
    q = q.astype(jnp.float32)
    nk = float(n * _K)
    sum_h = jnp.sum(h, axis=0, keepdims=True)
    sum_h2 = jnp.sum(h * h, axis=0, keepdims=True)
    sum_s = jnp.sum(s, axis=0, keepdims=True)
    cross = jnp.sum(h * s, axis=0, keepdims=True)
    sum_q = jnp.sum(q, axis=0, keepdims=True)
    mean = (_K * sum_h + sum_s) / nk
    e2 = (_K * sum_h2 + 2.0 * cross + sum_q) / nk
    inv = lax.rsqrt(e2 - mean * mean + 1e-5)
    v = (h + mx - mean) * inv
    return jnp.where(v >= 0, v, 0.2 * v)


def _stage1_body(n, h_ref, m_ref, s_ref, q_ref, wb_ref, wd_ref,
                 x1_ref, g2_ref, h2_ref):
    x1 = _stage_finish(h_ref[0], m_ref[0], s_ref[0], q_ref[0], n)
    x1_ref[0] = x1
    g2_ref[0] = jnp.dot(x1, wb_ref[...], preferred_element_type=jnp.float32)
    h2_ref[0] = jnp.dot(x1, wd_ref[...], preferred_element_type=jnp.float32)


def _stage1_call(h1, m1, s1, q1, w2bt, w2dt):
    b, n, c = h1.shape
    c2 = w2bt.shape[1]
    arr = lambda cc: pl.BlockSpec((1, n, cc), lambda bi: (bi, 0, 0))
    wspec = pl.BlockSpec((c, c2), lambda bi: (0, 0))
    return pl.pallas_call(
        functools.partial(_stage1_body, n),
        grid=(b,),
        in_specs=[arr(c), arr(c), arr(c), arr(c), wspec, wspec],
        out_specs=[arr(c), arr(c2), arr(c2)],
        out_shape=[
            jax.ShapeDtypeStruct((b, n, c), jnp.float32),
            jax.ShapeDtypeStruct((b, n, c2), jnp.float32),
            jax.ShapeDtypeStruct((b, n, c2), jnp.float32),
        ],
    )(h1, m1, s1, q1, w2bt, w2dt)


def _final_body(n, h2_ref, m2_ref, s2_ref, q2_ref, ft_ref, x1_ref,
                wa_ref, wb_ref, wc_ref, out_ref):
    x2 = _stage_finish(h2_ref[0], m2_ref[0], s2_ref[0], q2_ref[0], n)
    y = (jnp.dot(ft_ref[0], wa_ref[...], preferred_element_type=jnp.float32)
         + jnp.dot(x1_ref[0], wb_ref[...], preferred_element_type=jnp.float32)
         + jnp.dot(x2, wc_ref[...], preferred_element_type=jnp.float32))
    m3 = jnp.mean(y, axis=0, keepdims=True)
    e2 = jnp.mean(y * y, axis=0, keepdims=True)
    inv = lax.rsqrt(e2 - m3 * m3 + 1e-5)
    z = (y - m3) * inv
    z = jnp.where(z >= 0, z, 0.2 * z)
    out_ref[0] = z.T


def _final_call(h2, m2, s2, q2, ft, x1, w3at, w3bt, w3ct):
    b, n, c2 = h2.shape
    c = ft.shape[2]
    arr = lambda cc: pl.BlockSpec((1, n, cc), lambda bi: (bi, 0, 0))
    wsp = lambda ci: pl.BlockSpec((ci, c), lambda bi: (0, 0))
    return pl.pallas_call(
        functools.partial(_final_body, n),
        grid=(b,),
        in_specs=[arr(c2), arr(c2), arr(c2), arr(c2), arr(c), arr(c),
                  wsp(c), wsp(c), wsp(c2)],
        out_specs=pl.BlockSpec((1, c, n), lambda bi: (bi, 0, 0)),
        out_shape=jax.ShapeDtypeStruct((b, c, n), jnp.float32),
    )(h2, m2, s2, q2, ft, x1, w3at, w3bt, w3ct)


def kernel(coords, features, W1, W2, W3):
    b, c, n = features.shape
    ptsT = jnp.swapaxes(coords, 1, 2)
    ft = jnp.swapaxes(features, 1, 2)
    w1bt = W1[:, c:].T
    w1dt = (W1[:, :c] - W1[:, c:]).T
    w2bt = W2[:, c:].T
    w2dt = (W2[:, :c] - W2[:, c:]).T
    w3at = W3[:, :c].T
    w3bt = W3[:, c:2 * c].T
    w3ct = W3[:, 2 * c:].T

    g1, h1 = _proj_call(ft, w1bt, w1dt)
    nh = n // 2
    sc1 = _sc_gather_reduce(nh, c)
    sc2 = _sc_gather_reduce(n, 2 * c)

    outs = []
    for i in range(b):
        halves = []
        for h in range(2):
            ih = _topk_call(ptsT[i:i + 1, h * nh:(h + 1) * nh],
                            coords[i:i + 1]).reshape(-1)
            halves.append((ih, sc1(g1[i], ih)))
        idx_i = jnp.concatenate([hv[0] for hv in halves])
        m1, s1, q1 = (jnp.concatenate([hv[1][t] for hv in halves])
                      for t in range(3))
        x1, g2, h2 = _stage1_call(h1[i:i + 1], m1[None], s1[None], q1[None],
                                  w2bt, w2dt)
        m2, s2, q2 = sc2(g2[0], idx_i)
        outs.append(_final_call(h2, m2[None], s2[None], q2[None],
                                ft[i:i + 1], x1, w3at, w3bt, w3ct))
    return jnp.concatenate(outs, axis=0)


# speedup vs baseline: 1.1677x; 1.0513x over previous
import functools

import jax
import jax.numpy as jnp
from jax import lax
from jax.experimental import pallas as pl
from jax.experimental.pallas import tpu as pltpu
from jax.experimental.pallas import tpu_sc as plsc

_K = 16


def _topk_body(n, ptsT_ref, coords_ref, idx_ref):
    b = pl.program_id(0)
    q = ptsT_ref[0]
    c = coords_ref[0]
    sq_c = jnp.sum(c * c, axis=0, keepdims=True)
    sq_q = jnp.sum(q * q, axis=1, keepdims=True)
    qb = q.astype(jnp.bfloat16).astype(jnp.float32)
    cb = c.astype(jnp.bfloat16).astype(jnp.float32)
    prod = (qb[:, 0:1] * cb[0:1, :] + qb[:, 1:2] * cb[1:2, :]
            + qb[:, 2:3] * cb[2:3, :])
    d = sq_q + sq_c - 2.0 * prod
    iota = lax.broadcasted_iota(jnp.int32, d.shape, 1)
    inf = jnp.float32(jnp.inf)
    cols = []
    for j in range(_K + 1):
        am = jnp.argmin(d, axis=1).astype(jnp.int32)[:, None]
        if j > 0:
            cols.append(am + b * n)
        if j < _K:
            d = jnp.where(iota == am, inf, d)
    idx_ref[0] = jnp.concatenate(cols, axis=1)


def _topk_call(ptsT, coords, bq=512):
    b, nq, _ = ptsT.shape
    n = coords.shape[2]
    return pl.pallas_call(
        functools.partial(_topk_body, n),
        grid=(b, nq // bq),
        in_specs=[
            pl.BlockSpec((1, bq, 3), lambda bi, i: (bi, i, 0)),
            pl.BlockSpec((1, 3, n), lambda bi, i: (bi, 0, 0)),
        ],
        out_specs=pl.BlockSpec((1, bq, _K), lambda bi, i: (bi, i, 0)),
        out_shape=jax.ShapeDtypeStruct((b, nq, _K), jnp.int32),
    )(ptsT, coords)


def _proj_body(x_ref, w1_ref, w2_ref, o1_ref, o2_ref):
    x = x_ref[0]
    o1_ref[0] = jnp.dot(x, w1_ref[...], preferred_element_type=jnp.float32)
    o2_ref[0] = jnp.dot(x, w2_ref[...], preferred_element_type=jnp.float32)


def _proj_call(ft, w1t, w2t):
    b, n, c = ft.shape
    co = w1t.shape[1]
    return pl.pallas_call(
        _proj_body,
        grid=(b,),
        in_specs=[
            pl.BlockSpec((1, n, c), lambda bi: (bi, 0, 0)),
            pl.BlockSpec((c, co), lambda bi: (0, 0)),
            pl.BlockSpec((c, co), lambda bi: (0, 0)),
        ],
        out_specs=[
            pl.BlockSpec((1, n, co), lambda bi: (bi, 0, 0)),
            pl.BlockSpec((1, n, co), lambda bi: (bi, 0, 0)),
        ],
        out_shape=[jax.ShapeDtypeStruct((b, n, co), jnp.float32),
                   jax.ShapeDtypeStruct((b, n, co), jnp.float32)],
    )(ft, w1t, w2t)


def _sc_gather_reduce(bn, c):
    nw = 32
    npw = bn // nw
    gp = 8
    ch = 4096 // c
    mesh = plsc.VectorSubcoreMesh(core_axis_name="c", subcore_axis_name="s")
    out_sds = jax.ShapeDtypeStruct((bn, c), jnp.float32)

    @functools.partial(
        pl.kernel,
        out_type=(out_sds, out_sds, out_sds),
        mesh=mesh,
        scratch_types=[
            pltpu.VMEM((npw * _K,), jnp.int32),
            pltpu.VMEM((gp * _K, c), jnp.float32),
            pltpu.VMEM((gp * _K, c), jnp.float32),
            pltpu.VMEM((ch, c), jnp.float32),
            pltpu.VMEM((ch, c), jnp.float32),
            pltpu.VMEM((ch, c), jnp.float32),
            pltpu.SemaphoreType.DMA,
            pltpu.SemaphoreType.DMA,
        ],
    )
    def kern(g_hbm, idx_hbm, m_hbm, s_hbm, q_hbm,
             idx_v, rows0_v, rows1_v, m_v, s_v, q_v, sem0, sem1):
        wid = lax.axis_index("c") * 16 + lax.axis_index("s")
        base_pt = wid * npw
        pltpu.sync_copy(idx_hbm.at[pl.ds(base_pt * _K, npw * _K)], idx_v)
        ngroups = ch // gp

        def src(ci, gi):
            off = (ci * ch + gi * gp) * _K
            return g_hbm.at[idx_v.at[pl.ds(off, gp * _K)]]

        def reduce_group(gi, rows_v):
            for p in range(gp):
                row = gi * gp + p
                for j in range(c // 16):
                    sl = pl.ds(j * 16, 16)
                    r = rows_v[p * _K, sl]
                    mx = r
                    sm = r
                    qq = r * r
                    for i in range(1, _K):
                        r = rows_v[p * _K + i, sl]
                        mx = jnp.maximum(mx, r)
                        sm = sm + r
                        qq = qq + r * r
                    m_v[row, sl] = mx
                    s_v[row, sl] = sm
                    q_v[row, sl] = qq

        bufs = (rows0_v, rows1_v)
        sems = (sem0, sem1)
        nchunks = npw // ch

        def chunk_body(ci, carry):
            for g in range(ngroups):
                if g + 1 < ngroups:
                    pltpu.async_copy(
                        src(ci, g + 1), bufs[(g + 1) % 2], sems[(g + 1) % 2])
                else:
                    @pl.when(ci + 1 < nchunks)
                    def _():
                        pltpu.async_copy(src(ci + 1, 0), rows0_v, sem0)
                pltpu.make_async_copy(
                    src(ci, g), bufs[g % 2], sems[g % 2]).wait()
                reduce_group(g, bufs[g % 2])

            out_off = base_pt + ci * ch
            pltpu.sync_copy(m_v, m_hbm.at[pl.ds(out_off, ch)])
            pltpu.sync_copy(s_v, s_hbm.at[pl.ds(out_off, ch)])
            pltpu.sync_copy(q_v, q_hbm.at[pl.ds(out_off, ch)])
            return carry

        pltpu.async_copy(src(0, 0), rows0_v, sem0)
        lax.fori_loop(0, nchunks, chunk_body, 0)

    return kern


def _stage_finish(h, mx, s, q, n):
    mx = mx.astype(jnp.float32)
    s = s.astype(jnp.float32)
    q = q.astype(jnp.float32)
    nk = float(n * _K)
    sum_h = jnp.sum(h, axis=0, keepdims=True)
    sum_h2 = jnp.sum(h * h, axis=0, keepdims=True)
    sum_s = jnp.sum(s, axis=0, keepdims=True)
    cross = jnp.sum(h * s, axis=0, keepdims=True)
    sum_q = jnp.sum(q, axis=0, keepdims=True)
    mean = (_K * sum_h + sum_s) / nk
    e2 = (_K * sum_h2 + 2.0 * cross + sum_q) / nk
    inv = lax.rsqrt(e2 - mean * mean + 1e-5)
    v = (h + mx - mean) * inv
    return jnp.where(v >= 0, v, 0.2 * v)


def _stage1_body(n, h_ref, m_ref, s_ref, q_ref, wb_ref, wd_ref,
                 x1_ref, g2_ref, h2_ref):
    x1 = _stage_finish(h_ref[0], m_ref[0], s_ref[0], q_ref[0], n)
    x1_ref[0] = x1
    g2_ref[0] = jnp.dot(x1, wb_ref[...], preferred_element_type=jnp.float32)
    h2_ref[0] = jnp.dot(x1, wd_ref[...], preferred_element_type=jnp.float32)


def _stage1_call(h1, m1, s1, q1, w2bt, w2dt):
    b, n, c = h1.shape
    c2 = w2bt.shape[1]
    arr = lambda cc: pl.BlockSpec((1, n, cc), lambda bi: (bi, 0, 0))
    wspec = pl.BlockSpec((c, c2), lambda bi: (0, 0))
    return pl.pallas_call(
        functools.partial(_stage1_body, n),
        grid=(b,),
        in_specs=[arr(c), arr(c), arr(c), arr(c), wspec, wspec],
        out_specs=[arr(c), arr(c2), arr(c2)],
        out_shape=[
            jax.ShapeDtypeStruct((b, n, c), jnp.float32),
            jax.ShapeDtypeStruct((b, n, c2), jnp.float32),
            jax.ShapeDtypeStruct((b, n, c2), jnp.float32),
        ],
    )(h1, m1, s1, q1, w2bt, w2dt)


def _final_body(n, h2_ref, m2_ref, s2_ref, q2_ref, ft_ref, x1_ref,
                wa_ref, wb_ref, wc_ref, out_ref):
    x2 = _stage_finish(h2_ref[0], m2_ref[0], s2_ref[0], q2_ref[0], n)
    y = (jnp.dot(ft_ref[0], wa_ref[...], preferred_element_type=jnp.float32)
         + jnp.dot(x1_ref[0], wb_ref[...], preferred_element_type=jnp.float32)
         + jnp.dot(x2, wc_ref[...], preferred_element_type=jnp.float32))
    m3 = jnp.mean(y, axis=0, keepdims=True)
    e2 = jnp.mean(y * y, axis=0, keepdims=True)
    inv = lax.rsqrt(e2 - m3 * m3 + 1e-5)
    z = (y - m3) * inv
    z = jnp.where(z >= 0, z, 0.2 * z)
    out_ref[0] = z.T


def _final_call(h2, m2, s2, q2, ft, x1, w3at, w3bt, w3ct):
    b, n, c2 = h2.shape
    c = ft.shape[2]
    arr = lambda cc: pl.BlockSpec((1, n, cc), lambda bi: (bi, 0, 0))
    wsp = lambda ci: pl.BlockSpec((ci, c), lambda bi: (0, 0))
    return pl.pallas_call(
        functools.partial(_final_body, n),
        grid=(b,),
        in_specs=[arr(c2), arr(c2), arr(c2), arr(c2), arr(c), arr(c),
                  wsp(c), wsp(c), wsp(c2)],
        out_specs=pl.BlockSpec((1, c, n), lambda bi: (bi, 0, 0)),
        out_shape=jax.ShapeDtypeStruct((b, c, n), jnp.float32),
    )(h2, m2, s2, q2, ft, x1, w3at, w3bt, w3ct)


def kernel(coords, features, W1, W2, W3):
    b, c, n = features.shape
    ptsT = jnp.swapaxes(coords, 1, 2)
    ft = jnp.swapaxes(features, 1, 2)
    w1bt = W1[:, c:].T
    w1dt = (W1[:, :c] - W1[:, c:]).T
    w2bt = W2[:, c:].T
    w2dt = (W2[:, :c] - W2[:, c:]).T
    w3at = W3[:, :c].T
    w3bt = W3[:, c:2 * c].T
    w3ct = W3[:, 2 * c:].T

    g1, h1 = _proj_call(ft, w1bt, w1dt)
    nh = n // 2
    sc1 = _sc_gather_reduce(nh, c)
    sc2 = _sc_gather_reduce(n, 2 * c)

    outs = []
    for i in range(b):
        halves = []
        for h in range(2):
            ih = _topk_call(ptsT[i:i + 1, h * nh:(h + 1) * nh],
                            coords[i:i + 1]).reshape(-1)
            halves.append((ih, sc1(g1[i], ih)))
        idx_i = jnp.concatenate([hv[0] for hv in halves])
        m1, s1, q1 = (jnp.concatenate([hv[1][t] for hv in halves])
                      for t in range(3))
        x1, g2, h2 = _stage1_call(h1[i:i + 1], m1[None], s1[None], q1[None],
                                  w2bt, w2dt)
        m2, s2, q2 = sc2(g2[0], idx_i)
        outs.append(_final_call(h2, m2[None], s2[None], q2[None],
                                ft[i:i + 1], x1, w3at, w3bt, w3ct))
    return jnp.concatenate(outs, axis=0)
